# baseline (speedup 1.0000x reference)
import jax
import jax.numpy as jnp
from jax import lax
from jax.experimental import pallas as pl
from jax.experimental.pallas import tpu as pltpu
from jax.experimental.pallas import tpu_sc as plsc

_MAX_RANKS = 50
_BATCH = 16384
_LIST_SIZE = 50
_TAB = 64
_NC = 2
_NS = 16
_NW = _NC * _NS
_COLS_W = _BATCH // _NW
_L = 16
_NQ = 4
_CQ = _COLS_W // _NQ
_KQ = _CQ // _L


def _sc_body(x_hbm, w_hbm, out_hbm, x0_v, x1_v, x2_v, x3_v, y_v, w_v, tab_v,
             sem0, sem1, sem2, sem3, sem_out):
    wid = lax.axis_index("s") * _NC + lax.axis_index("c")
    c0 = wid * _COLS_W

    xq = (x0_v, x1_v, x2_v, x3_v)
    sems = (sem0, sem1, sem2, sem3)
    ins = [
        pltpu.async_copy(x_hbm.at[:, pl.ds(c0 + q * _CQ, _CQ)], xq[q], sems[q])
        for q in range(_NQ)
    ]

    pltpu.sync_copy(w_hbm, w_v)
    zeros = jnp.zeros((_L,), jnp.float32)
    for s in (0, 16, 32, 48):
        tab_v[pl.ds(s, _L)] = zeros
    iota = lax.iota(jnp.int32, _L)
    for s in (0, 16, 32, 34):
        plsc.store_scatter(tab_v, [iota + (s + 1)], w_v[pl.ds(s, _L)])

    for q in range(_NQ):
        ins[q].wait()
        x_v = xq[q]

        @plsc.parallel_loop(0, _LIST_SIZE * _KQ, 1, unroll=16)
        def win(i, x_v=x_v, q=q):
            r = lax.shift_right_logical(i, 3)
            s = lax.bitwise_and(i, _KQ - 1) * _L
            x = x_v[r, pl.ds(s, _L)]
            j = lax.bitwise_and(x, _TAB - 1)
            y_v[pl.ds(r * _COLS_W + q * _CQ + s, _L)] = (
                plsc.load_gather(tab_v, [j]))

        if q % 2 == 1:
            h = q // 2

            def emit(r, carry, h=h):
                pltpu.async_copy(
                    y_v.at[pl.ds(r * _COLS_W + h * 2 * _CQ, 2 * _CQ)],
                    out_hbm.at[pl.ds(r * _BATCH + c0 + h * 2 * _CQ, 2 * _CQ)],
                    sem_out,
                )
                return carry

            lax.fori_loop(0, _LIST_SIZE, emit, 0)

    pltpu.make_async_copy(
        out_hbm.at[pl.ds(0, _LIST_SIZE * _COLS_W)],
        y_v,
        sem_out,
    ).wait()


def kernel(inputs, W):
    call = pl.kernel(
        _sc_body,
        out_type=jax.ShapeDtypeStruct((_BATCH * _LIST_SIZE,), jnp.float32),
        mesh=plsc.VectorSubcoreMesh(core_axis_name="c", subcore_axis_name="s"),
        compiler_params=pltpu.CompilerParams(
            needs_layout_passes=False, skip_device_barrier=True),
        scratch_types=[
            pltpu.VMEM((_LIST_SIZE, _CQ), jnp.int32),
            pltpu.VMEM((_LIST_SIZE, _CQ), jnp.int32),
            pltpu.VMEM((_LIST_SIZE, _CQ), jnp.int32),
            pltpu.VMEM((_LIST_SIZE, _CQ), jnp.int32),
            pltpu.VMEM((_LIST_SIZE * _COLS_W,), jnp.float32),
            pltpu.VMEM((_MAX_RANKS,), jnp.float32),
            pltpu.VMEM((_TAB,), jnp.float32),
            pltpu.SemaphoreType.DMA,
            pltpu.SemaphoreType.DMA,
            pltpu.SemaphoreType.DMA,
            pltpu.SemaphoreType.DMA,
            pltpu.SemaphoreType.DMA,
        ],
    )
    out = call(inputs.T, W.reshape(_MAX_RANKS))
    return out.reshape(_LIST_SIZE, 1, _BATCH).transpose(2, 0, 1)


# speedup vs baseline: 1.0070x; 1.0070x over previous
import jax
import jax.numpy as jnp
from jax import lax
from jax.experimental import pallas as pl
from jax.experimental.pallas import tpu as pltpu
from jax.experimental.pallas import tpu_sc as plsc

_MAX_RANKS = 50
_BATCH = 16384
_LIST_SIZE = 50
_TAB = 64
_NC = 2
_NS = 16
_NW = _NC * _NS
_COLS_W = _BATCH // _NW
_L = 16
_NQ = 4
_CQ = _COLS_W // _NQ
_KQ = _CQ // _L


def _sc_body(x_hbm, w_hbm, out_hbm, x0_v, x1_v, x2_v, x3_v, y_v, w_v, tab_v,
             sem0, sem1, sem2, sem3, sem_out):
    wid = lax.axis_index("s") * _NC + lax.axis_index("c")
    c0 = wid * _COLS_W

    xq = (x0_v, x1_v, x2_v, x3_v)
    sems = (sem0, sem1, sem2, sem3)
    ins = [
        pltpu.async_copy(x_hbm.at[:, pl.ds(c0 + q * _CQ, _CQ)], xq[q], sems[q])
        for q in range(_NQ)
    ]

    pltpu.sync_copy(w_hbm, w_v)
    zeros = jnp.zeros((_L,), jnp.float32)
    for s in (0, 16, 32, 48):
        tab_v[pl.ds(s, _L)] = zeros
    iota = lax.iota(jnp.int32, _L)
    for s in (0, 16, 32, 34):
        plsc.store_scatter(tab_v, [iota + (s + 1)], w_v[pl.ds(s, _L)])

    for q in range(_NQ):
        ins[q].wait()
        x_v = xq[q]

        @plsc.parallel_loop(0, _LIST_SIZE * _KQ, 1, unroll=8)
        def win(i, x_v=x_v, q=q):
            r = lax.shift_right_logical(i, 3)
            s = lax.bitwise_and(i, _KQ - 1) * _L
            x = x_v[r, pl.ds(s, _L)]
            j = lax.bitwise_and(x, _TAB - 1)
            y_v[pl.ds(r * _COLS_W + q * _CQ + s, _L)] = (
                plsc.load_gather(tab_v, [j]))

        if q % 2 == 1:
            h = q // 2

            def emit(r, carry, h=h):
                pltpu.async_copy(
                    y_v.at[pl.ds(r * _COLS_W + h * 2 * _CQ, 2 * _CQ)],
                    out_hbm.at[pl.ds(r * _BATCH + c0 + h * 2 * _CQ, 2 * _CQ)],
                    sem_out,
                )
                return carry

            lax.fori_loop(0, _LIST_SIZE, emit, 0)

    pltpu.make_async_copy(
        out_hbm.at[pl.ds(0, _LIST_SIZE * _COLS_W)],
        y_v,
        sem_out,
    ).wait()


def kernel(inputs, W):
    call = pl.kernel(
        _sc_body,
        out_type=jax.ShapeDtypeStruct((_BATCH * _LIST_SIZE,), jnp.float32),
        mesh=plsc.VectorSubcoreMesh(core_axis_name="c", subcore_axis_name="s"),
        compiler_params=pltpu.CompilerParams(
            needs_layout_passes=False, skip_device_barrier=True),
        scratch_types=[
            pltpu.VMEM((_LIST_SIZE, _CQ), jnp.int32),
            pltpu.VMEM((_LIST_SIZE, _CQ), jnp.int32),
            pltpu.VMEM((_LIST_SIZE, _CQ), jnp.int32),
            pltpu.VMEM((_LIST_SIZE, _CQ), jnp.int32),
            pltpu.VMEM((_LIST_SIZE * _COLS_W,), jnp.float32),
            pltpu.VMEM((_MAX_RANKS,), jnp.float32),
            pltpu.VMEM((_TAB,), jnp.float32),
            pltpu.SemaphoreType.DMA,
            pltpu.SemaphoreType.DMA,
            pltpu.SemaphoreType.DMA,
            pltpu.SemaphoreType.DMA,
            pltpu.SemaphoreType.DMA,
        ],
    )
    out = call(inputs.T, W.reshape(_MAX_RANKS))
    return out.reshape(_LIST_SIZE, 1, _BATCH).transpose(2, 0, 1)
